# baseline (speedup 1.0000x reference)
import functools

import jax
import jax.numpy as jnp
from jax import lax
from jax.experimental import pallas as pl
from jax.experimental.pallas import tpu as pltpu
from jax.experimental.pallas import tpu_sc as plsc

_VOCAB = 1000000
_DIM = 32
_BATCH = 4096
_HIST = 200

_N = _BATCH * _HIST
_NC = 2
_NS = 16
_NW = _NC * _NS
_PER_W = _N // _NW
_CHUNK = 1024
_NSUB = _CHUNK // 128
_NCHUNK = _PER_W // _CHUNK
_LANES = 16
_RPB = 128 // _DIM


def _rsqrt16(x):
    i = plsc.bitcast(x, jnp.int32)
    i = jnp.int32(0x5F3759DF) - lax.shift_right_logical(i, 1)
    y = plsc.bitcast(i, jnp.float32)
    for _ in range(3):
        y = y * (jnp.float32(1.5) - jnp.float32(0.5) * x * y * y)
    return y


@functools.partial(
    pl.kernel,
    mesh=plsc.VectorSubcoreMesh(core_axis_name="c", subcore_axis_name="s"),
    out_type=jax.ShapeDtypeStruct((_N, _DIM), jnp.float32),
    scratch_types=[
        pltpu.VMEM((_NSUB, 128), jnp.int32),
        pltpu.VMEM((_CHUNK, _DIM), jnp.float32),
        pltpu.SemaphoreType.DMA,
    ],
    compiler_params=pltpu.CompilerParams(
        use_tc_tiling_on_sc=False, needs_layout_passes=False
    ),
)
def _gather_normalize(idx_hbm, table_hbm, out_hbm, idx_v, rows_v, sem):
    cid = lax.axis_index("c")
    sid = lax.axis_index("s")
    wid = sid * _NC + cid
    row0_w = wid * _PER_W

    lane = lax.iota(jnp.int32, _LANES)

    def chunk_body(g, carry):
        base = pl.multiple_of(row0_w + g * _CHUNK, _CHUNK)
        pltpu.sync_copy(
            idx_hbm.at[pl.ds(pl.multiple_of(base // 128, 8), _NSUB)], idx_v
        )
        copies = [
            pltpu.async_copy(
                table_hbm.at[idx_v.at[j]],
                rows_v.at[pl.ds(j * 128, 128)],
                sem,
            )
            for j in range(_NSUB)
        ]
        for c in copies:
            c.wait()

        def norm_body(i, carry2):
            r = i * _LANES + lane
            acc = jnp.zeros((_LANES,), jnp.float32)
            vals = []
            for d in range(_DIM):
                col = jnp.full((_LANES,), d, jnp.int32)
                v = plsc.load_gather(rows_v, [r, col])
                acc = acc + v * v
                vals.append(v)
            scale = jnp.minimum(_rsqrt16(acc), jnp.float32(1e12))
            for d in range(_DIM):
                col = jnp.full((_LANES,), d, jnp.int32)
                plsc.store_scatter(rows_v, [r, col], vals[d] * scale)
            return carry2

        lax.fori_loop(0, _CHUNK // _LANES, norm_body, 0)

        pltpu.sync_copy(rows_v, out_hbm.at[pl.ds(base, _CHUNK)])
        return carry

    lax.fori_loop(0, _NCHUNK, chunk_body, 0)


def kernel(input, W):
    idx = jnp.transpose(input, (1, 0)).reshape(_N // 128, 128)
    idx = idx.astype(jnp.int32)
    out = _gather_normalize(idx, W)
    return out.reshape(_HIST, _BATCH, _DIM)


# speedup vs baseline: 1.6823x; 1.6823x over previous
import functools

import jax
import jax.numpy as jnp
from jax import lax
from jax.experimental import pallas as pl
from jax.experimental.pallas import tpu as pltpu
from jax.experimental.pallas import tpu_sc as plsc

_VOCAB = 1000000
_DIM = 32
_BATCH = 4096
_HIST = 200

_N = _BATCH * _HIST
_NC = 2
_NS = 16
_NW = _NC * _NS
_CHUNK = 1024
_NSUB = _CHUNK // 128
_NQ = _BATCH // _CHUNK
_NTILE = _HIST * _NQ
_PER_W = _NTILE // _NW
_LANES = 16
_DBLK = _DIM // 8
_BBLK = _BATCH // 128


def _rsqrt16(x):
    i = plsc.bitcast(x, jnp.int32)
    i = jnp.int32(0x5F3759DF) - lax.shift_right_logical(i, 1)
    y = plsc.bitcast(i, jnp.float32)
    for _ in range(3):
        y = y * (jnp.float32(1.5) - jnp.float32(0.5) * x * y * y)
    return y


@functools.partial(
    pl.kernel,
    mesh=plsc.VectorSubcoreMesh(core_axis_name="c", subcore_axis_name="s"),
    out_type=jax.ShapeDtypeStruct((_HIST, _DBLK, _BBLK, 8, 128), jnp.float32),
    scratch_types=[
        pltpu.VMEM((_NSUB, 128), jnp.int32),
        pltpu.VMEM((_CHUNK, _DIM), jnp.float32),
        pltpu.VMEM((_DBLK, _NSUB, 8, 128), jnp.float32),
        pltpu.SemaphoreType.DMA,
    ],
    compiler_params=pltpu.CompilerParams(
        use_tc_tiling_on_sc=False, needs_layout_passes=False
    ),
)
def _gather_normalize(idx_hbm, table_hbm, out_hbm, idx_v, rows_v, stage_v, sem):
    cid = lax.axis_index("c")
    sid = lax.axis_index("s")
    wid = sid * _NC + cid
    tile0 = wid * _PER_W

    lane = lax.iota(jnp.int32, _LANES)

    def tile_body(k, carry):
        t = tile0 + k
        h = t // _NQ
        q = t % _NQ
        pltpu.sync_copy(
            idx_hbm.at[h, pl.ds(pl.multiple_of(q * _NSUB, 8), _NSUB)], idx_v
        )
        copies = [
            pltpu.async_copy(
                table_hbm.at[idx_v.at[j]],
                rows_v.at[pl.ds(j * 128, 128)],
                sem,
            )
            for j in range(_NSUB)
        ]
        for c in copies:
            c.wait()

        def norm_body(i, carry2):
            r = i * _LANES + lane
            bblk = i // 8
            boff = (i % 8) * _LANES
            acc = jnp.zeros((_LANES,), jnp.float32)
            vals = []
            for d in range(_DIM):
                col = jnp.full((_LANES,), d, jnp.int32)
                v = plsc.load_gather(rows_v, [r, col])
                acc = acc + v * v
                vals.append(v)
            scale = jnp.minimum(_rsqrt16(acc), jnp.float32(1e12))
            for d in range(_DIM):
                stage_v[d // 8, bblk, d % 8, pl.ds(boff, _LANES)] = (
                    vals[d] * scale
                )
            return carry2

        lax.fori_loop(0, _CHUNK // _LANES, norm_body, 0)

        pltpu.sync_copy(
            stage_v,
            out_hbm.at[h, :, pl.ds(pl.multiple_of(q * _NSUB, 8), _NSUB)],
        )
        return carry

    lax.fori_loop(0, _PER_W, tile_body, 0)


def kernel(input, W):
    idx = jnp.transpose(input, (1, 0)).reshape(_HIST, _BATCH // 128, 128)
    idx = idx.astype(jnp.int32)
    out5 = _gather_normalize(idx, W)
    return out5.transpose(0, 2, 4, 1, 3).reshape(_HIST, _BATCH, _DIM)


# speedup vs baseline: 1.8663x; 1.1093x over previous
import functools

import jax
import jax.numpy as jnp
from jax import lax
from jax.experimental import pallas as pl
from jax.experimental.pallas import tpu as pltpu
from jax.experimental.pallas import tpu_sc as plsc

_VOCAB = 1000000
_DIM = 32
_BATCH = 4096
_HIST = 200

_N = _BATCH * _HIST
_NC = 2
_NS = 16
_NW = _NC * _NS
_CHUNK = 512
_NSUB = _CHUNK // 128
_NQ = _BATCH // _CHUNK
_NTILE = _HIST * _NQ
_PER_W = _NTILE // _NW
_IDXROW = _N // _NW // 128
_LANES = 16
_DBLK = _DIM // 8


def _rsqrt16(x):
    i = plsc.bitcast(x, jnp.int32)
    i = jnp.int32(0x5F3759DF) - lax.shift_right_logical(i, 1)
    y = plsc.bitcast(i, jnp.float32)
    for _ in range(3):
        y = y * (jnp.float32(1.5) - jnp.float32(0.5) * x * y * y)
    return y


@functools.partial(
    pl.kernel,
    mesh=plsc.VectorSubcoreMesh(core_axis_name="c", subcore_axis_name="s"),
    out_type=jax.ShapeDtypeStruct((_HIST, _DBLK, _BATCH // 128, 8, 128), jnp.float32),
    scratch_types=[
        pltpu.VMEM((_IDXROW, 128), jnp.int32),
        pltpu.VMEM((2, _CHUNK, _DIM), jnp.float32),
        pltpu.VMEM((2, _DBLK, _NSUB, 8, 128), jnp.float32),
        pltpu.SemaphoreType.DMA,
        pltpu.SemaphoreType.DMA,
        pltpu.SemaphoreType.DMA,
        pltpu.SemaphoreType.DMA,
    ],
    compiler_params=pltpu.CompilerParams(
        use_tc_tiling_on_sc=False, needs_layout_passes=False
    ),
)
def _gather_normalize(
    idx_hbm, table_hbm, out_hbm, idx_v, rows_v, stage_v,
    sem_g0, sem_g1, sem_o0, sem_o1,
):
    cid = lax.axis_index("c")
    sid = lax.axis_index("s")
    wid = sid * _NC + cid
    chunk0 = wid * _PER_W
    sem_g = (sem_g0, sem_g1)
    sem_o = (sem_o0, sem_o1)

    lane = lax.iota(jnp.int32, _LANES)

    pltpu.sync_copy(
        idx_hbm.at[pl.ds(pl.multiple_of(wid * _IDXROW, 8), _IDXROW)], idx_v
    )

    def fire_gathers(k, b):
        for j in range(_NSUB):
            pltpu.async_copy(
                table_hbm.at[idx_v.at[k * _NSUB + j]],
                rows_v.at[b, pl.ds(j * 128, 128)],
                sem_g[b],
            )

    def wait_gathers(k, b):
        for j in range(_NSUB):
            pltpu.make_async_copy(
                table_hbm.at[idx_v.at[k * _NSUB + j]],
                rows_v.at[b, pl.ds(j * 128, 128)],
                sem_g[b],
            ).wait()

    def out_slice(k):
        t = chunk0 + k
        h = t // _NQ
        q = t % _NQ
        return out_hbm.at[
            h, :, pl.ds(pl.multiple_of(q * _NSUB, _NSUB), _NSUB)
        ]

    def normalize(k, b):
        def norm_body(i, carry):
            r = i * _LANES + lane
            bblk = i // 8
            boff = (i % 8) * _LANES
            acc = jnp.zeros((_LANES,), jnp.float32)
            vals = []
            for d in range(_DIM):
                col = jnp.full((_LANES,), d, jnp.int32)
                v = plsc.load_gather(rows_v.at[b], [r, col])
                acc = acc + v * v
                vals.append(v)
            scale = jnp.minimum(_rsqrt16(acc), jnp.float32(1e12))
            for d in range(_DIM):
                stage_v[b, d // 8, bblk, d % 8, pl.ds(boff, _LANES)] = (
                    vals[d] * scale
                )
            return carry

        lax.fori_loop(0, _CHUNK // _LANES, norm_body, 0)

    fire_gathers(0, 0)
    fire_gathers(1, 1)

    def body2(k2, carry):
        for b in range(2):
            k = k2 * 2 + b
            wait_gathers(k, b)

            @pl.when(k2 >= 1)
            def _():
                pltpu.make_async_copy(
                    stage_v.at[b], out_slice(k), sem_o[b]
                ).wait()

            normalize(k, b)
            pltpu.async_copy(stage_v.at[b], out_slice(k), sem_o[b])

            @pl.when(k2 < (_PER_W // 2 - 1))
            def _():
                fire_gathers(k + 2, b)
        return carry

    lax.fori_loop(0, _PER_W // 2, body2, 0)

    for b in range(2):
        pltpu.make_async_copy(
            stage_v.at[b], out_slice(_PER_W - 2 + b), sem_o[b]
        ).wait()


def kernel(input, W):
    idx = jnp.transpose(input, (1, 0)).reshape(_N // 128, 128)
    idx = idx.astype(jnp.int32)
    out5 = _gather_normalize(idx, W)
    return out5.transpose(0, 2, 4, 1, 3).reshape(_HIST, _BATCH, _DIM)
